# baseline (speedup 1.0000x reference)
import functools

import jax
import jax.numpy as jnp
from jax import lax
from jax.experimental import pallas as pl
from jax.experimental.pallas import tpu as pltpu
from jax.experimental.pallas import tpu_sc as plsc

VOCAB = 100000
D = 1024
B = 4
L = 4096
BL = B * L

NC = 2
NS = 16
NLANES = 16
NW = NC * NS
LW = L // NW
C = 32
NLC = LW // C
VPR = D // NLANES


def _body(ids_hbm, tok_hbm, pos_hbm, out_hbm, idx_v, pos_v, tok_v, sem):
    wid = lax.axis_index("s") * NC + lax.axis_index("c")
    l_base = wid * LW

    def pos_chunk(lc, _):
        l_off = l_base + lc * C
        pltpu.sync_copy(pos_hbm.at[pl.ds(l_off, C)], pos_v)

        def batch_body(b, _):
            off = b * L + l_off
            pltpu.sync_copy(ids_hbm.at[pl.ds(off, C)], idx_v)
            pltpu.async_copy(tok_hbm.at[idx_v], tok_v, sem).wait()

            def row_body(r, _):
                def vec_body(j, _):
                    v = pos_v[r, pl.ds(j * NLANES, NLANES)]
                    plsc.addupdate(tok_v.at[r, pl.ds(j * NLANES, NLANES)], v)
                    return _
                return lax.fori_loop(0, VPR, vec_body, _)

            lax.fori_loop(0, C, row_body, 0)
            pltpu.sync_copy(tok_v, out_hbm.at[pl.ds(off, C)])
            return _

        return lax.fori_loop(0, B, batch_body, _)

    lax.fori_loop(0, NLC, pos_chunk, 0)


@jax.jit
def _embed(ids_flat, token_embed, pos_embed):
    mesh = plsc.VectorSubcoreMesh(
        core_axis_name="c", subcore_axis_name="s", num_cores=NC, num_subcores=NS
    )
    f = pl.kernel(
        _body,
        out_type=jax.ShapeDtypeStruct((BL, D), jnp.float32),
        mesh=mesh,
        scratch_types=[
            pltpu.VMEM((C,), jnp.int32),
            pltpu.VMEM((C, D), jnp.float32),
            pltpu.VMEM((C, D), jnp.float32),
            pltpu.SemaphoreType.DMA,
        ],
    )
    return f(ids_flat, token_embed, pos_embed)


def kernel(input_ids, token_embed, pos_embed):
    ids_flat = input_ids.reshape(-1).astype(jnp.int32)
    out = _embed(ids_flat, token_embed, pos_embed)
    return out.reshape(B, L, D)


# speedup vs baseline: 1.9071x; 1.9071x over previous
import functools

import jax
import jax.numpy as jnp
from jax import lax
from jax.experimental import pallas as pl
from jax.experimental.pallas import tpu as pltpu
from jax.experimental.pallas import tpu_sc as plsc

VOCAB = 100000
D = 1024
B = 4
L = 4096
BL = B * L

NC = 2
NS = 16
NLANES = 16
NW = NC * NS
LW = L // NW
C = 32
NLC = LW // C
VPR = D // NLANES


def _body(ids_hbm, tok_hbm, pos_hbm, out_hbm, idx_v, pos_v, tok_v, sem):
    wid = lax.axis_index("s") * NC + lax.axis_index("c")
    l_base = wid * LW

    def pos_chunk(lc, _):
        l_off = l_base + lc * C
        pltpu.sync_copy(pos_hbm.at[pl.ds(l_off, C)], pos_v)

        def batch_body(b, _):
            off = b * L + l_off
            pltpu.sync_copy(ids_hbm.at[pl.ds(off, C)], idx_v)
            pltpu.async_copy(tok_hbm.at[idx_v], tok_v, sem).wait()

            @plsc.parallel_loop(0, C * VPR, unroll=8)
            def _add(i):
                r = lax.shift_right_logical(i, 6)
                col = pl.multiple_of(
                    lax.shift_left(jnp.bitwise_and(i, VPR - 1), 4), NLANES
                )
                v = pos_v[r, pl.ds(col, NLANES)]
                plsc.addupdate(tok_v.at[r, pl.ds(col, NLANES)], v)
            pltpu.sync_copy(tok_v, out_hbm.at[pl.ds(off, C)])
            return _

        return lax.fori_loop(0, B, batch_body, _)

    lax.fori_loop(0, NLC, pos_chunk, 0)


@jax.jit
def _embed(ids_flat, token_embed, pos_embed):
    mesh = plsc.VectorSubcoreMesh(
        core_axis_name="c", subcore_axis_name="s", num_cores=NC, num_subcores=NS
    )
    f = pl.kernel(
        _body,
        out_type=jax.ShapeDtypeStruct((BL, D), jnp.float32),
        mesh=mesh,
        scratch_types=[
            pltpu.VMEM((C,), jnp.int32),
            pltpu.VMEM((C, D), jnp.float32),
            pltpu.VMEM((C, D), jnp.float32),
            pltpu.SemaphoreType.DMA,
        ],
    )
    return f(ids_flat, token_embed, pos_embed)


def kernel(input_ids, token_embed, pos_embed):
    ids_flat = input_ids.reshape(-1).astype(jnp.int32)
    out = _embed(ids_flat, token_embed, pos_embed)
    return out.reshape(B, L, D)


# speedup vs baseline: 2.9870x; 1.5663x over previous
import functools

import jax
import jax.numpy as jnp
from jax import lax
from jax.experimental import pallas as pl
from jax.experimental.pallas import tpu as pltpu
from jax.experimental.pallas import tpu_sc as plsc

VOCAB = 100000
D = 1024
B = 4
L = 4096
BL = B * L

NC = 2
NS = 16
NLANES = 16
NW = NC * NS
LW = L // NW
C = 16
NLC = LW // C
VPR = D // NLANES
NCH = NLC * B
KT = 4


def _body(ids_hbm, tok_hbm, pos_hbm, out_hbm,
          idx_v, pos0, pos1, t0, t1, t2, t3, gsem, ssem, psem):
    toks = (t0, t1, t2, t3)
    poss = (pos0, pos1)
    wid = lax.axis_index("s") * NC + lax.axis_index("c")
    l_base = wid * LW

    for b in range(B):
        pltpu.sync_copy(ids_hbm.at[b, pl.ds(l_base, LW)], idx_v.at[b])

    def start_pos(lc):
        return pltpu.async_copy(
            pos_hbm.at[pl.ds(l_base + lc * C, C)], poss[lc % 2],
            psem.at[lc % 2])

    def start_gather(g):
        lc, b = divmod(g, B)
        s = g % KT
        idx = idx_v.at[b, pl.ds(lc * C, C)]
        return pltpu.async_copy(tok_hbm.at[idx], toks[s], gsem.at[s])

    def start_store(g):
        lc, b = divmod(g, B)
        s = g % KT
        off = b * L + l_base + lc * C
        return pltpu.async_copy(toks[s], out_hbm.at[pl.ds(off, C)],
                                ssem.at[s])

    def add(g):
        lc = g // B
        tok = toks[g % KT]
        pos = poss[lc % 2]

        @plsc.parallel_loop(0, C * VPR, unroll=8)
        def _add(i):
            r = lax.shift_right_logical(i, 6)
            col = pl.multiple_of(
                lax.shift_left(jnp.bitwise_and(i, VPR - 1), 4), NLANES)
            plsc.addupdate(tok.at[r, pl.ds(col, NLANES)],
                           pos[r, pl.ds(col, NLANES)])

    pos_descs = [None] * NLC
    g_descs = [None] * NCH
    s_descs = [None] * NCH

    pos_descs[0] = start_pos(0)
    pos_descs[1] = start_pos(1)
    g_descs[0] = start_gather(0)
    g_descs[1] = start_gather(1)

    for g in range(NCH):
        lc, b = divmod(g, B)
        if g + 2 < NCH:
            if g - 2 >= 0:
                s_descs[g - 2].wait()
            g_descs[g + 2] = start_gather(g + 2)
        if b == 0:
            pos_descs[lc].wait()
        g_descs[g].wait()
        add(g)
        if b == B - 1 and lc + 2 < NLC:
            pos_descs[lc + 2] = start_pos(lc + 2)
        s_descs[g] = start_store(g)

    for g in range(NCH - 4, NCH):
        s_descs[g].wait()


@jax.jit
def _embed(input_ids, token_embed, pos_embed):
    mesh = plsc.VectorSubcoreMesh(
        core_axis_name="c", subcore_axis_name="s", num_cores=NC, num_subcores=NS
    )
    f = pl.kernel(
        _body,
        out_type=jax.ShapeDtypeStruct((BL, D), jnp.float32),
        mesh=mesh,
        scratch_types=[
            pltpu.VMEM((B, LW), jnp.int32),
            pltpu.VMEM((C, D), jnp.float32),
            pltpu.VMEM((C, D), jnp.float32),
            pltpu.VMEM((C, D), jnp.float32),
            pltpu.VMEM((C, D), jnp.float32),
            pltpu.VMEM((C, D), jnp.float32),
            pltpu.VMEM((C, D), jnp.float32),
            pltpu.SemaphoreType.DMA((KT,)),
            pltpu.SemaphoreType.DMA((KT,)),
            pltpu.SemaphoreType.DMA((2,)),
        ],
    )
    return f(input_ids, token_embed, pos_embed)


def kernel(input_ids, token_embed, pos_embed):
    out = _embed(input_ids.astype(jnp.int32), token_embed, pos_embed)
    return out.reshape(B, L, D)
